# Initial kernel scaffold; baseline (speedup 1.0000x reference)
import jax
import jax.numpy as jnp
from jax.experimental import pallas as pl


def kernel(x, W1, b1, W2, b2):
    raise NotImplementedError("write your pallas kernel here")


# speedup vs baseline: 116.3151x; 116.3151x over previous
import jax
import jax.numpy as jnp
from jax.experimental import pallas as pl

_H = 768
_E = 64


def _router_body(x_ref, w1_ref, b1_ref, w2_ref, b2_ref, out_ref):
    x0 = x_ref[0]
    h = jax.lax.dot_general(
        x0, w1_ref[...], (((1,), (1,)), ((), ())),
        preferred_element_type=jnp.float32)
    h = jnp.maximum(h + b1_ref[...], 0.0)
    logits = jax.lax.dot_general(
        h, w2_ref[...], (((1,), (1,)), ((), ())),
        preferred_element_type=jnp.float32)
    logits = logits + b2_ref[...]

    ids = jax.lax.broadcasted_iota(jnp.int32, (1, _E), 1)
    v1 = jnp.max(logits, axis=1, keepdims=True)
    i1 = jnp.min(jnp.where(logits == v1, ids, _E), axis=1, keepdims=True)
    rest = jnp.where(ids == i1, -jnp.inf, logits)
    v2 = jnp.max(rest, axis=1, keepdims=True)
    i2 = jnp.min(jnp.where(rest == v2, ids, _E), axis=1, keepdims=True)

    e2 = jnp.exp(v2 - v1)
    denom = 1.0 + e2
    out_ref[...] = jnp.where(
        ids == i1, 1.0 / denom, jnp.where(ids == i2, e2 / denom, 0.0))


def kernel(x, W1, b1, W2, b2):
    out = pl.pallas_call(
        _router_body,
        grid=(1,),
        in_specs=[
            pl.BlockSpec((1, 1, _H), lambda i: (0, 0, 0)),
            pl.BlockSpec((_H, _H), lambda i: (0, 0)),
            pl.BlockSpec((1, _H), lambda i: (0, 0)),
            pl.BlockSpec((_E, _H), lambda i: (0, 0)),
            pl.BlockSpec((1, _E), lambda i: (0, 0)),
        ],
        out_specs=pl.BlockSpec((1, _E), lambda i: (0, 0)),
        out_shape=jax.ShapeDtypeStruct((1, _E), jnp.float32),
    )(x, W1, b1.reshape(1, _H), W2, b2.reshape(1, _E))
    return out.reshape(_E)
